# baseline (speedup 1.0000x reference)
import functools

import jax
import jax.numpy as jnp
from jax.experimental import pallas as pl
from jax.experimental.pallas import tpu as pltpu

N = 1024
D = 64
B = 8
IN_F = 2 * D
HID = 2 * IN_F
NPAIR = 2 * B
GRP = 4
CAP = 128
NCHUNK = N // CAP


def _mlp_rows(x, W1, b1, g, beta, W2, b2, W3, b3, W4r, b4):
    h = jnp.maximum(jnp.dot(x, W1, preferred_element_type=jnp.float32) + b1, 0.0)
    mu = jnp.mean(h, axis=-1, keepdims=True)
    var = jnp.mean((h - mu) ** 2, axis=-1, keepdims=True)
    h = (h - mu) * jax.lax.rsqrt(var + 1e-5) * g + beta
    h = jnp.maximum(jnp.dot(h, W2, preferred_element_type=jnp.float32) + b2, 0.0)
    h = jnp.maximum(jnp.dot(h, W3, preferred_element_type=jnp.float32) + b3, 0.0)
    return jnp.sum(h * W4r, axis=-1, keepdims=True) + b4


def _body(counts_ref, nodes_ref, adj_ref, E_ref,
          nW1, nb1, ng, nbeta, nW2, nb2, nW3, nb3, nW4r, nb4,
          oW1, ob1, og, obeta, oW2, ob2, oW3, ob3, oW4r, ob4,
          out_ref, contrib_ref, adjf_ref):
    (nW1, nb1, ng, nbeta, nW2, nb2, nW3, nb3, nW4r, nb4,
     oW1, ob1, og, obeta, oW2, ob2, oW3, ob3, oW4r, ob4) = (
        r[...] for r in (nW1, nb1, ng, nbeta, nW2, nb2, nW3, nb3, nW4r, nb4,
                         oW1, ob1, og, obeta, oW2, ob2, oW3, ob3, oW4r, ob4))
    E = E_ref[...]
    KB = 256
    for rb in range(N // KB):
        adjf_ref[rb * KB:(rb + 1) * KB, :] = (
            adj_ref[rb * KB:(rb + 1) * KB, :].astype(jnp.float32))
    adjf = adjf_ref[...]

    nodes = nodes_ref[0:1, :]
    row_ids = jax.lax.broadcasted_iota(jnp.int32, (N, NPAIR), 0)
    onehot = (row_ids == nodes).astype(jnp.float32)

    nbcols = jnp.dot(adjf, onehot, preferred_element_type=jnp.float32)
    erows = jax.lax.dot_general(onehot, E, (((0,), (0,)), ((), ())),
                                preferred_element_type=jnp.float32)
    nb_tar = nbcols[:, 0:B]
    nb_src = nbcols[:, B:NPAIR]
    nbrows = jax.lax.dot_general(onehot, adjf, (((0,), (0,)), ((), ())),
                                 preferred_element_type=jnp.float32)
    nbt_r = nbrows[0:B, :]
    nbs_r = nbrows[B:NPAIR, :]
    wmask_t = jnp.concatenate([nbs_r * (1.0 - nbt_r),
                               (1.0 - nbs_r) * nbt_r], axis=0)

    tri_r = jax.lax.broadcasted_iota(jnp.int32, (N, N), 0)
    tri_c = jax.lax.broadcasted_iota(jnp.int32, (N, N), 1)
    triu = (tri_r <= tri_c).astype(jnp.float32)
    pos_t = jnp.dot(wmask_t, triu, preferred_element_type=jnp.float32)
    key_t = 2.0 * pos_t - 1.0 - wmask_t

    lane_ids = jax.lax.broadcasted_iota(jnp.int32, (D, GRP * D), 1)
    d_ids = jax.lax.broadcasted_iota(jnp.int32, (D, GRP * D), 0)
    tile_m = (lane_ids % D == d_ids).astype(jnp.float32)
    et4 = jnp.dot(E, tile_m, preferred_element_type=jnp.float32)

    krow0 = jax.lax.broadcasted_iota(jnp.int32, (CAP, N), 0).astype(jnp.float32)
    krow1 = jax.lax.broadcasted_iota(
        jnp.int32, (N - CAP, N), 0).astype(jnp.float32) + float(CAP)

    for gidx in range(NPAIR // GRP):
        pairs = range(gidx * GRP, (gidx + 1) * GRP)
        pr_ids = jax.lax.broadcasted_iota(jnp.int32, (NPAIR, GRP * D), 0)
        col_ids = jax.lax.broadcasted_iota(jnp.int32, (NPAIR, GRP * D), 1)
        exp_g = (pr_ids == gidx * GRP + col_ids // D).astype(jnp.float32)
        me = jnp.dot(nbcols, exp_g, preferred_element_type=jnp.float32) * et4
        cn = jnp.zeros((N, GRP * D), jnp.float32)
        for kb in range(N // KB):
            cn += jnp.dot(adjf[:, kb * KB:(kb + 1) * KB],
                          me[kb * KB:(kb + 1) * KB, :],
                          preferred_element_type=jnp.float32)
        for j, p in enumerate(pairs):
            cn_p = cn[:, j * D:(j + 1) * D]
            e_other = erows[p:p + 1, :]
            keyb = key_t[p:p + 1, :]

            def chunk(krow, keyb=keyb, cn_p=cn_p, e_other=e_other):
                s_t = (keyb == 2.0 * krow).astype(jnp.float32)
                ec = jnp.dot(s_t, E, preferred_element_type=jnp.float32)
                cnc = jnp.dot(s_t, cn_p, preferred_element_type=jnp.float32)
                xc = jnp.concatenate([ec * e_other, cnc], axis=1)
                logit = _mlp_rows(xc, nW1, nb1, ng, nbeta, nW2, nb2, nW3,
                                  nb3, nW4r, nb4)
                a = jax.nn.sigmoid(logit)
                return jax.lax.dot_general(a, ec, (((0,), (0,)), ((), ())),
                                           preferred_element_type=jnp.float32)

            contrib_ref[p:p + 1, :] = chunk(krow0)
            @pl.when(counts_ref[p] > CAP)
            def _():
                contrib_ref[p:p + 1, :] += chunk(krow1)

    contrib = contrib_ref[...]
    both = nb_src * nb_tar
    both_e = jax.lax.dot_general(both, E, (((0,), (0,)), ((), ())),
                                 preferred_element_type=jnp.float32)
    all_cn = both_e + contrib[0:B, :] + contrib[B:NPAIR, :]
    prod = erows[B:NPAIR, :] * erows[0:B, :]
    final = jnp.concatenate([prod, all_cn], axis=1)
    out_ref[...] = _mlp_rows(final, oW1, ob1, og, obeta, oW2, ob2, oW3, ob3,
                             oW4r, ob4)


@jax.jit
def _run(counts, nodes, adjf, E, *weights):
    full = lambda a: pl.BlockSpec(a.shape, lambda: (0,) * a.ndim)
    args = (nodes, adjf, E) + weights
    return pl.pallas_call(
        _body,
        out_shape=jax.ShapeDtypeStruct((B, 1), jnp.float32),
        in_specs=[pl.BlockSpec(memory_space=pltpu.SMEM)] + [full(a) for a in args],
        out_specs=pl.BlockSpec((B, 1), lambda: (0, 0)),
        scratch_shapes=[pltpu.VMEM((NPAIR, D), jnp.float32),
                        pltpu.VMEM((N, N), jnp.float32)],
    )(counts, *args)


def kernel(src, dst, adjacent, NodeEmbedding,
           ncn_W1, ncn_b1, ncn_g, ncn_beta, ncn_W2, ncn_b2, ncn_W3, ncn_b3,
           ncn_W4, ncn_b4,
           out_W1, out_b1, out_g, out_beta, out_W2, out_b2, out_W3, out_b3,
           out_W4, out_b4):
    nodes = jnp.broadcast_to(
        jnp.concatenate([dst, src]).reshape(1, NPAIR), (8, NPAIR))
    nbs = adjacent[src, :]
    nbt = adjacent[dst, :]
    counts = jnp.concatenate([
        jnp.sum(nbs & ~nbt, axis=1, dtype=jnp.int32),
        jnp.sum(~nbs & nbt, axis=1, dtype=jnp.int32)])
    r2 = lambda v: v.reshape(1, -1)
    weights = (
        ncn_W1, r2(ncn_b1), r2(ncn_g), r2(ncn_beta), ncn_W2, r2(ncn_b2),
        ncn_W3, r2(ncn_b3), ncn_W4.reshape(1, HID), r2(ncn_b4),
        out_W1, r2(out_b1), r2(out_g), r2(out_beta), out_W2, r2(out_b2),
        out_W3, r2(out_b3), out_W4.reshape(1, HID), r2(out_b4),
    )
    return _run(counts, nodes, adjacent, NodeEmbedding, *weights)


# speedup vs baseline: 1.1519x; 1.1519x over previous
import functools

import jax
import jax.numpy as jnp
from jax.experimental import pallas as pl
from jax.experimental.pallas import tpu as pltpu

N = 1024
D = 64
B = 8
IN_F = 2 * D
HID = 2 * IN_F
NPAIR = 2 * B
GRP = 4
CAP = 128
NCHUNK = N // CAP


def _mlp_rows(x, W1, b1, g, beta, W2, b2, W3, b3, W4r, b4):
    h = jnp.maximum(jnp.dot(x, W1, preferred_element_type=jnp.float32) + b1, 0.0)
    mu = jnp.mean(h, axis=-1, keepdims=True)
    var = jnp.mean((h - mu) ** 2, axis=-1, keepdims=True)
    h = (h - mu) * jax.lax.rsqrt(var + 1e-5) * g + beta
    h = jnp.maximum(jnp.dot(h, W2, preferred_element_type=jnp.float32) + b2, 0.0)
    h = jnp.maximum(jnp.dot(h, W3, preferred_element_type=jnp.float32) + b3, 0.0)
    return jnp.sum(h * W4r, axis=-1, keepdims=True) + b4


def _body(counts_ref, nodes_ref, adj_ref, E_ref,
          nW1, nb1, ng, nbeta, nW2, nb2, nW3, nb3, nW4r, nb4,
          oW1, ob1, og, obeta, oW2, ob2, oW3, ob3, oW4r, ob4,
          out_ref, contrib_ref, adjf_ref):
    (nW1, nb1, ng, nbeta, nW2, nb2, nW3, nb3, nW4r, nb4,
     oW1, ob1, og, obeta, oW2, ob2, oW3, ob3, oW4r, ob4) = (
        r[...] for r in (nW1, nb1, ng, nbeta, nW2, nb2, nW3, nb3, nW4r, nb4,
                         oW1, ob1, og, obeta, oW2, ob2, oW3, ob3, oW4r, ob4))
    E = E_ref[...]
    KB = 256
    for rb in range(N // KB):
        adjf_ref[rb * KB:(rb + 1) * KB, :] = (
            adj_ref[rb * KB:(rb + 1) * KB, :].astype(jnp.float32))
    adjf = adjf_ref[...]

    nodes = nodes_ref[0:1, :]
    row_ids = jax.lax.broadcasted_iota(jnp.int32, (N, NPAIR), 0)
    onehot = (row_ids == nodes).astype(jnp.float32)

    nbcols = jnp.dot(adjf, onehot, preferred_element_type=jnp.float32)
    erows = jax.lax.dot_general(onehot, E, (((0,), (0,)), ((), ())),
                                preferred_element_type=jnp.float32)
    nb_tar = nbcols[:, 0:B]
    nb_src = nbcols[:, B:NPAIR]
    nbrows = jax.lax.dot_general(onehot, adjf, (((0,), (0,)), ((), ())),
                                 preferred_element_type=jnp.float32)
    nbt_r = nbrows[0:B, :]
    nbs_r = nbrows[B:NPAIR, :]
    wmask_t = jnp.concatenate([nbs_r * (1.0 - nbt_r),
                               (1.0 - nbs_r) * nbt_r], axis=0)

    tri_r = jax.lax.broadcasted_iota(jnp.int32, (N, N), 0)
    tri_c = jax.lax.broadcasted_iota(jnp.int32, (N, N), 1)
    triu = (tri_r <= tri_c).astype(jnp.float32)
    pos_t = jnp.dot(wmask_t, triu, preferred_element_type=jnp.float32)
    key_t = 2.0 * pos_t - 1.0 - wmask_t

    lane_ids = jax.lax.broadcasted_iota(jnp.int32, (D, GRP * D), 1)
    d_ids = jax.lax.broadcasted_iota(jnp.int32, (D, GRP * D), 0)
    tile_m = (lane_ids % D == d_ids).astype(jnp.float32)
    et4 = jnp.dot(E, tile_m, preferred_element_type=jnp.float32)

    krow0 = jax.lax.broadcasted_iota(jnp.int32, (CAP, N), 0).astype(jnp.float32)
    krow1 = jax.lax.broadcasted_iota(
        jnp.int32, (N - CAP, N), 0).astype(jnp.float32) + float(CAP)

    ecs, xcs, cns = [], [], []
    for gidx in range(NPAIR // GRP):
        pairs = range(gidx * GRP, (gidx + 1) * GRP)
        pr_ids = jax.lax.broadcasted_iota(jnp.int32, (NPAIR, GRP * D), 0)
        col_ids = jax.lax.broadcasted_iota(jnp.int32, (NPAIR, GRP * D), 1)
        exp_g = (pr_ids == gidx * GRP + col_ids // D).astype(jnp.float32)
        me = jnp.dot(nbcols, exp_g, preferred_element_type=jnp.float32) * et4
        cn = jnp.zeros((N, GRP * D), jnp.float32)
        for kb in range(N // KB):
            cn += jnp.dot(adjf[:, kb * KB:(kb + 1) * KB],
                          me[kb * KB:(kb + 1) * KB, :],
                          preferred_element_type=jnp.float32)
        s_g = jnp.concatenate(
            [(key_t[p:p + 1, :] == 2.0 * krow0).astype(jnp.float32)
             for p in pairs], axis=0)
        ec_g = jnp.dot(s_g, E, preferred_element_type=jnp.float32)
        cnc_g = jnp.dot(s_g, cn, preferred_element_type=jnp.float32)
        ecs.append(ec_g)
        xcs.append(jnp.concatenate(
            [jnp.concatenate(
                [ec_g[j * CAP:(j + 1) * CAP, :] * erows[p:p + 1, :],
                 cnc_g[j * CAP:(j + 1) * CAP, j * D:(j + 1) * D]], axis=1)
             for j, p in enumerate(pairs)], axis=0))
        cns.append(cn)

    xc_all = jnp.concatenate(xcs, axis=0)
    a_all = jax.nn.sigmoid(_mlp_rows(xc_all, nW1, nb1, ng, nbeta, nW2, nb2,
                                     nW3, nb3, nW4r, nb4))
    for p in range(NPAIR):
        ec_p = ecs[p // GRP][(p % GRP) * CAP:(p % GRP + 1) * CAP, :]
        a_p = a_all[p * CAP:(p + 1) * CAP, :]
        contrib_ref[p:p + 1, :] = jax.lax.dot_general(
            a_p, ec_p, (((0,), (0,)), ((), ())),
            preferred_element_type=jnp.float32)

        @pl.when(counts_ref[p] > CAP)
        def _(p=p):
            cn_p = cns[p // GRP][:, (p % GRP) * D:(p % GRP + 1) * D]
            s_t = (key_t[p:p + 1, :] == 2.0 * krow1).astype(jnp.float32)
            ec = jnp.dot(s_t, E, preferred_element_type=jnp.float32)
            cnc = jnp.dot(s_t, cn_p, preferred_element_type=jnp.float32)
            xc = jnp.concatenate([ec * erows[p:p + 1, :], cnc], axis=1)
            logit = _mlp_rows(xc, nW1, nb1, ng, nbeta, nW2, nb2, nW3,
                              nb3, nW4r, nb4)
            a = jax.nn.sigmoid(logit)
            contrib_ref[p:p + 1, :] += jax.lax.dot_general(
                a, ec, (((0,), (0,)), ((), ())),
                preferred_element_type=jnp.float32)

    contrib = contrib_ref[...]
    both = nb_src * nb_tar
    both_e = jax.lax.dot_general(both, E, (((0,), (0,)), ((), ())),
                                 preferred_element_type=jnp.float32)
    all_cn = both_e + contrib[0:B, :] + contrib[B:NPAIR, :]
    prod = erows[B:NPAIR, :] * erows[0:B, :]
    final = jnp.concatenate([prod, all_cn], axis=1)
    out_ref[...] = _mlp_rows(final, oW1, ob1, og, obeta, oW2, ob2, oW3, ob3,
                             oW4r, ob4)


@jax.jit
def _run(counts, nodes, adjf, E, *weights):
    full = lambda a: pl.BlockSpec(a.shape, lambda: (0,) * a.ndim)
    args = (nodes, adjf, E) + weights
    return pl.pallas_call(
        _body,
        out_shape=jax.ShapeDtypeStruct((B, 1), jnp.float32),
        in_specs=[pl.BlockSpec(memory_space=pltpu.SMEM)] + [full(a) for a in args],
        out_specs=pl.BlockSpec((B, 1), lambda: (0, 0)),
        scratch_shapes=[pltpu.VMEM((NPAIR, D), jnp.float32),
                        pltpu.VMEM((N, N), jnp.float32)],
    )(counts, *args)


def kernel(src, dst, adjacent, NodeEmbedding,
           ncn_W1, ncn_b1, ncn_g, ncn_beta, ncn_W2, ncn_b2, ncn_W3, ncn_b3,
           ncn_W4, ncn_b4,
           out_W1, out_b1, out_g, out_beta, out_W2, out_b2, out_W3, out_b3,
           out_W4, out_b4):
    nodes = jnp.broadcast_to(
        jnp.concatenate([dst, src]).reshape(1, NPAIR), (8, NPAIR))
    nbs = adjacent[src, :]
    nbt = adjacent[dst, :]
    counts = jnp.concatenate([
        jnp.sum(nbs & ~nbt, axis=1, dtype=jnp.int32),
        jnp.sum(~nbs & nbt, axis=1, dtype=jnp.int32)])
    r2 = lambda v: v.reshape(1, -1)
    weights = (
        ncn_W1, r2(ncn_b1), r2(ncn_g), r2(ncn_beta), ncn_W2, r2(ncn_b2),
        ncn_W3, r2(ncn_b3), ncn_W4.reshape(1, HID), r2(ncn_b4),
        out_W1, r2(out_b1), r2(out_g), r2(out_beta), out_W2, r2(out_b2),
        out_W3, r2(out_b3), out_W4.reshape(1, HID), r2(out_b4),
    )
    return _run(counts, nodes, adjacent, NodeEmbedding, *weights)
